# Initial kernel scaffold; baseline (speedup 1.0000x reference)
import jax
import jax.numpy as jnp
from jax.experimental import pallas as pl


def kernel(data, emb0, emb1):
    raise NotImplementedError("write your pallas kernel here")


# speedup vs baseline: 1.5910x; 1.5910x over previous
import functools

import jax
import jax.numpy as jnp
from jax import lax
from jax.experimental import pallas as pl
from jax.experimental.pallas import tpu as pltpu
from jax.experimental.pallas import tpu_sc as plsc

_VOCAB = 1000000
_D = 64
_NEG = 5
_B = 16384

_NC = 2
_NS = 16
_NW = _NC * _NS
_ROWS_PER_TILE = _B // _NW
_C = 128
_NCHUNK = _ROWS_PER_TILE // _C
_GROUPS = _C // 16


def _softplus(z):
    t = jnp.exp(-jnp.abs(z))
    s = t / (t + 2.0)
    s2 = s * s
    p = jnp.float32(1.0 / 9.0)
    p = p * s2 + jnp.float32(1.0 / 7.0)
    p = p * s2 + jnp.float32(1.0 / 5.0)
    p = p * s2 + jnp.float32(1.0 / 3.0)
    p = p * s2 + jnp.float32(1.0)
    return jnp.maximum(z, 0.0) + 2.0 * s * p


def _sc_body(widx_hbm, e1idx_hbm, mask_hbm, emb0_hbm, emb1_hbm, out_hbm,
             idx0_v, idx1_v, mask_v, w_v, e1_v, out_v, sem_a, sem_b):
    wid = lax.axis_index("s") * _NC + lax.axis_index("c")

    pltpu.sync_copy(widx_hbm.at[wid], idx0_v)
    pltpu.sync_copy(e1idx_hbm.at[wid], idx1_v)
    pltpu.sync_copy(mask_hbm.at[wid], mask_v)

    sems = [sem_a, sem_b]

    def issue(c, buf):
        cps = [pltpu.async_copy(emb0_hbm.at[idx0_v.at[c]], w_v.at[buf],
                                sems[buf])]
        for j in range(6):
            cps.append(pltpu.async_copy(emb1_hbm.at[idx1_v.at[c * 6 + j]],
                                        e1_v.at[buf, j], sems[buf]))
        return cps

    lane = lax.iota(jnp.int32, 16)

    def compute_chunk(c, buf, accs):
        wbuf = w_v.at[buf]
        ebuf = e1_v.at[buf]

        def gbody(g, accs):
            acc_pos, acc_neg = accs
            rloc = lane + g * 16
            rt = rloc + c * 128
            flat6 = rloc * 6
            js = []
            iss = []
            for k in range(6):
                f = flat6 + k
                js.append(lax.shift_right_logical(f, 7))
                iss.append(lax.bitwise_and(f, 127))

            def dbody(d, dots):
                dvec = jnp.full((16,), d, dtype=jnp.int32)
                wv = plsc.load_gather(wbuf, [rloc, dvec])
                cv = plsc.load_gather(ebuf, [js[0], iss[0], dvec])
                new = [dots[0] + wv * cv]
                for k in range(1, 6):
                    nv = plsc.load_gather(ebuf, [js[k], iss[k], dvec])
                    new.append(dots[k] + nv * wv)
                return tuple(new)

            zero = jnp.zeros((16,), jnp.float32)
            dots = lax.fori_loop(0, _D, dbody, (zero,) * 6)

            acc_pos = acc_pos + _softplus(-dots[0])
            for k in range(1, 6):
                mvec = plsc.load_gather(mask_v, [rt * _NEG + (k - 1)])
                acc_neg = acc_neg + _softplus(dots[k] * mvec)
            return (acc_pos, acc_neg)

        return lax.fori_loop(0, _GROUPS, gbody, accs)

    zero = jnp.zeros((16,), jnp.float32)
    accs = (zero, zero)
    descs = [None, None]
    descs[0] = issue(0, 0)
    for c in range(_NCHUNK):
        buf = c % 2
        if c + 1 < _NCHUNK:
            descs[(c + 1) % 2] = issue(c + 1, (c + 1) % 2)
        for d in descs[buf]:
            d.wait()
        accs = compute_chunk(c, buf, accs)

    out_v[0, :] = accs[0]
    out_v[1, :] = accs[1]
    pltpu.sync_copy(out_v, out_hbm.at[wid])


_mesh = plsc.VectorSubcoreMesh(core_axis_name="c", subcore_axis_name="s",
                               num_cores=_NC, num_subcores=_NS)

_sc_kernel = functools.partial(
    pl.kernel,
    out_type=jax.ShapeDtypeStruct((_NW, 2, 16), jnp.float32),
    mesh=_mesh,
    compiler_params=pltpu.CompilerParams(needs_layout_passes=False,
                                         use_tc_tiling_on_sc=False),
    scratch_types=[
        pltpu.VMEM((_NCHUNK, _C), jnp.int32),
        pltpu.VMEM((_NCHUNK * 6, _C), jnp.int32),
        pltpu.VMEM((_ROWS_PER_TILE * _NEG,), jnp.float32),
        pltpu.VMEM((2, _C, _D), jnp.float32),
        pltpu.VMEM((2, 6, _C, _D), jnp.float32),
        pltpu.VMEM((2, 16), jnp.float32),
        pltpu.SemaphoreType.DMA,
        pltpu.SemaphoreType.DMA,
    ],
)(_sc_body)


def kernel(data, emb0, emb1):
    data = data.astype(jnp.int32)
    widx = data[:, 0].reshape(_NW, _NCHUNK, _C)
    e1idx = data[:, 1:2 + _NEG].reshape(_NW, _NCHUNK * 6, _C)
    maskf = data[:, 2 + _NEG:].astype(jnp.float32).reshape(
        _NW, _ROWS_PER_TILE * _NEG)
    out = _sc_kernel(widx, e1idx, maskf, emb0, emb1)
    pos_loss = jnp.sum(out[:, 0, :])
    neg_loss = jnp.sum(out[:, 1, :])
    return (pos_loss, neg_loss)
